# baseline (speedup 1.0000x reference)
import functools

import jax
import jax.numpy as jnp
from jax import lax
from jax.experimental import pallas as pl
from jax.experimental.pallas import tpu as pltpu
from jax.experimental.pallas import tpu_sc as plsc

N = 10000
E = 320000
D = 128
NUM_LAYERS = 4
TRI = (0, 1, 3, 6)

NC = 2
NS = 16
NW = NC * NS

EPW = E // NW
CHUNK = 80

CPB = 32
EBLK = CPB * CHUNK
NBLK = -(-EPW // EBLK)
PAD = NBLK * EBLK - EPW

NACC = 10240
ROWS_PER_TILE = NACC // NS
TRASH = N + 8

BM = 2000


def _matmul_body(x_ref, w_ref, h_ref):
    h_ref[0] = jnp.dot(x_ref[0], w_ref[0], preferred_element_type=jnp.float32)


def _build_h(xs_stack, w_all, t):
    grid = (t + 1, N // BM)
    return pl.pallas_call(
        _matmul_body,
        grid=grid,
        in_specs=[
            pl.BlockSpec((1, BM, D), lambda j, i, t=t: (t - j, i, 0)),
            pl.BlockSpec((1, D, D), lambda j, i, t=t: (TRI[t] + j, 0, 0)),
        ],
        out_specs=pl.BlockSpec((1, BM, D), lambda j, i: (j, i, 0)),
        out_shape=jax.ShapeDtypeStruct((NUM_LAYERS, N, D), jnp.float32),
    )(xs_stack, w_all)


def _edge_body(t, h_ref, attr_ref, src_ref, dst_ref, p_ref,
               a_v, s_v, d_v, sgb, sdb, rows0, rows1, acc,
               lsem, gsem0, gsem1):
    c = lax.axis_index("c")
    s = lax.axis_index("s")
    wid = c * NS + s
    base0 = pl.multiple_of(wid * EPW, 16)

    zero16 = jnp.zeros((16,), jnp.float32)
    def zrow(i, carry):
        for j in range(D // 16):
            rows0[i, pl.ds(j * 16, 16)] = zero16
        return carry
    lax.fori_loop(0, CHUNK, zrow, 0)

    row0 = s * ROWS_PER_TILE
    def zacc(r, carry):
        pltpu.sync_copy(rows0, acc.at[pl.ds(row0 + r * CHUNK, CHUNK)])
        return carry
    lax.fori_loop(0, ROWS_PER_TILE // CHUNK, zacc, 0)
    plsc.subcore_barrier()

    iota = lax.broadcasted_iota(jnp.int32, (16,), 0)

    def gather(ci, rows, sem):
        return pltpu.async_copy(h_ref.at[sgb.at[ci]], rows, sem)

    def wait_gather(rows, sem):
        pltpu.make_async_copy(h_ref.at[sgb.at[0]], rows, sem).wait()

    def scatter(ci, rows):
        pltpu.sync_copy(rows, acc.at[sdb.at[ci]], add=True)

    for blk in range(NBLK):
        bbase = pl.multiple_of(base0 + blk * EBLK, 16)
        la = pltpu.async_copy(attr_ref.at[pl.ds(bbase, EBLK)], a_v, lsem)
        ls = pltpu.async_copy(src_ref.at[pl.ds(bbase, EBLK)], s_v, lsem)
        ld = pltpu.async_copy(dst_ref.at[pl.ds(bbase, EBLK)], d_v, lsem)
        la.wait()
        ls.wait()
        ld.wait()

        last = blk == NBLK - 1
        all_attrs = t == NUM_LAYERS - 1

        def chunk_idx(ci, carry, blk=blk, last=last, all_attrs=all_attrs):
            for m in range(CHUNK // 16):
                sl = pl.ds(ci * CHUNK + m * 16, 16)
                a = a_v[sl]
                sg = (a - 1) * N + s_v[sl]
                sd = d_v[sl]
                if not (all_attrs and not last):
                    if all_attrs:
                        act = None
                    else:
                        act = a <= (t + 1)
                    if last:
                        pos = blk * EBLK + ci * CHUNK + m * 16 + iota
                        pm = pos < EPW
                        act = pm if act is None else (act & pm)
                    sg = jnp.where(act, sg, 0)
                    sd = jnp.where(act, sd, TRASH)
                sl2 = pl.ds(m * 16, 16)
                sgb[ci, sl2] = sg
                sdb[ci, sl2] = sd
            return carry
        lax.fori_loop(0, CPB, chunk_idx, 0)

        gather(0, rows0, gsem0)

        def pair(j, carry):
            c0 = 2 * j
            gather(c0 + 1, rows1, gsem1)
            wait_gather(rows0, gsem0)
            scatter(c0, rows0)
            gather(c0 + 2, rows0, gsem0)
            wait_gather(rows1, gsem1)
            scatter(c0 + 1, rows1)
            return carry
        lax.fori_loop(0, CPB // 2 - 1, pair, 0)
        gather(CPB - 1, rows1, gsem1)
        wait_gather(rows0, gsem0)
        scatter(CPB - 2, rows0)
        wait_gather(rows1, gsem1)
        scatter(CPB - 1, rows1)

    plsc.subcore_barrier()
    pltpu.sync_copy(acc.at[pl.ds(row0, ROWS_PER_TILE)],
                    p_ref.at[c, pl.ds(row0, ROWS_PER_TILE)])


def _edge_pass(h2, attr_p, src_p, dst_p, t):
    mesh = plsc.VectorSubcoreMesh(core_axis_name="c", subcore_axis_name="s",
                                  num_cores=NC, num_subcores=NS)
    kern = pl.kernel(
        functools.partial(_edge_body, t),
        out_type=jax.ShapeDtypeStruct((NC, NACC, D), jnp.float32),
        mesh=mesh,
        scratch_types=[
            pltpu.VMEM((EBLK,), jnp.int32),
            pltpu.VMEM((EBLK,), jnp.int32),
            pltpu.VMEM((EBLK,), jnp.int32),
            pltpu.VMEM((CPB, CHUNK), jnp.int32),
            pltpu.VMEM((CPB, CHUNK), jnp.int32),
            pltpu.VMEM((CHUNK, D), jnp.float32),
            pltpu.VMEM((CHUNK, D), jnp.float32),
            pltpu.VMEM_SHARED((NACC, D), jnp.float32),
            pltpu.SemaphoreType.DMA,
            pltpu.SemaphoreType.DMA,
            pltpu.SemaphoreType.DMA,
        ],
    )
    return kern(h2, attr_p, src_p, dst_p)


def _combine_body(xs_ref, p_ref, o_ref):
    o_ref[0] = xs_ref[0] + jnp.maximum(p_ref[0] + p_ref[1], 0.0)


def _combine_next(xs_stack, p, t):
    return pl.pallas_call(
        _combine_body,
        grid=(N // BM,),
        in_specs=[
            pl.BlockSpec((1, BM, D), lambda i, t=t: (t, i, 0)),
            pl.BlockSpec((NC, BM, D), lambda i: (0, i, 0)),
        ],
        out_specs=pl.BlockSpec((1, BM, D), lambda i, t=t: (t + 1, i, 0)),
        out_shape=jax.ShapeDtypeStruct((NUM_LAYERS, N, D), jnp.float32),
        input_output_aliases={0: 0},
    )(xs_stack, p)


def _combine_final_body(xs_ref, p_ref, o_ref):
    o_ref[...] = xs_ref[0] + jnp.maximum(p_ref[0] + p_ref[1], 0.0)


def _combine_final(xs_stack, p):
    return pl.pallas_call(
        _combine_final_body,
        grid=(N // BM,),
        in_specs=[
            pl.BlockSpec((1, BM, D), lambda i: (NUM_LAYERS - 1, i, 0)),
            pl.BlockSpec((NC, BM, D), lambda i: (0, i, 0)),
        ],
        out_specs=pl.BlockSpec((BM, D), lambda i: (i, 0)),
        out_shape=jax.ShapeDtypeStruct((N, D), jnp.float32),
    )(xs_stack, p)


def kernel(x, edge_index, edge_attr, W):
    src = edge_index[0]
    dst = edge_index[1]
    zpad = jnp.zeros((PAD,), jnp.int32)
    attr_p = jnp.concatenate([edge_attr, zpad])
    src_p = jnp.concatenate([src, zpad])
    dst_p = jnp.concatenate([dst, zpad])
    xs_stack = jnp.zeros((NUM_LAYERS, N, D), jnp.float32).at[0].set(x)
    out = None
    for t in range(NUM_LAYERS):
        h = _build_h(xs_stack, W, t)
        h2 = h.reshape(NUM_LAYERS * N, D)
        p = _edge_pass(h2, attr_p, src_p, dst_p, t)
        if t < NUM_LAYERS - 1:
            xs_stack = _combine_next(xs_stack, p, t)
        else:
            out = _combine_final(xs_stack, p)
    return out


# speedup vs baseline: 30.2112x; 30.2112x over previous
import functools

import jax
import jax.numpy as jnp
from jax import lax
from jax.experimental import pallas as pl
from jax.experimental.pallas import tpu as pltpu
from jax.experimental.pallas import tpu_sc as plsc

N = 10000
E = 320000
D = 128
NUM_LAYERS = 4
TRI = (0, 1, 3, 6)

NC = 2
NS = 16
NW = NC * NS

EPW = E // NW
CHUNK = 80

CPB = 32
EBLK = CPB * CHUNK
NBLK = -(-EPW // EBLK)
PAD = NBLK * EBLK - EPW

NACC = 10240
ROWS_PER_TILE = NACC // NS

BM = 2000


def _matmul_body(x_ref, w_ref, h_ref):
    h_ref[0] = jnp.dot(x_ref[0], w_ref[0], preferred_element_type=jnp.float32)


def _build_h(xs_stack, w_all, t):
    grid = (t + 1, N // BM)
    return pl.pallas_call(
        _matmul_body,
        grid=grid,
        in_specs=[
            pl.BlockSpec((1, BM, D), lambda j, i, t=t: (t - j, i, 0)),
            pl.BlockSpec((1, D, D), lambda j, i, t=t: (TRI[t] + j, 0, 0)),
        ],
        out_specs=pl.BlockSpec((1, BM, D), lambda j, i: (j, i, 0)),
        out_shape=jax.ShapeDtypeStruct((NUM_LAYERS, N, D), jnp.float32),
    )(xs_stack, w_all)


def _edge_body(t, h_ref, attr_ref, src_ref, dst_ref, p_ref,
               a_v, s_v, d_v, sgb, sdb, rows0, rows1, acc,
               lsem, gsem0, gsem1):
    c = lax.axis_index("c")
    s = lax.axis_index("s")
    wid = c * NS + s
    base0 = pl.multiple_of(wid * EPW, 16)

    zero16 = jnp.zeros((16,), jnp.float32)
    def zrow(i, carry):
        for j in range(D // 16):
            rows0[i, pl.ds(j * 16, 16)] = zero16
        return carry
    lax.fori_loop(0, CHUNK, zrow, 0)

    row0 = s * ROWS_PER_TILE
    def zacc(r, carry):
        pltpu.sync_copy(rows0, acc.at[pl.ds(row0 + r * CHUNK, CHUNK)])
        return carry
    lax.fori_loop(0, ROWS_PER_TILE // CHUNK, zacc, 0)
    plsc.subcore_barrier()

    iota = lax.broadcasted_iota(jnp.int32, (16,), 0)

    def gather(ci, rows, sem):
        return pltpu.async_copy(h_ref.at[sgb.at[ci]], rows, sem)

    def wait_gather(rows, sem):
        pltpu.make_async_copy(h_ref.at[sgb.at[0]], rows, sem).wait()

    def scatter(ci, rows):
        pltpu.sync_copy(rows, acc.at[sdb.at[ci]], add=True)

    for blk in range(NBLK):
        bbase = pl.multiple_of(base0 + blk * EBLK, 16)
        la = pltpu.async_copy(attr_ref.at[pl.ds(bbase, EBLK)], a_v, lsem)
        ls = pltpu.async_copy(src_ref.at[pl.ds(bbase, EBLK)], s_v, lsem)
        ld = pltpu.async_copy(dst_ref.at[pl.ds(bbase, EBLK)], d_v, lsem)
        la.wait()
        ls.wait()
        ld.wait()

        last = blk == NBLK - 1
        all_attrs = t == NUM_LAYERS - 1

        def compute(ci, blk=blk, last=last, all_attrs=all_attrs):
            for m in range(CHUNK // 16):
                sl = pl.ds(ci * CHUNK + m * 16, 16)
                a = a_v[sl]
                sv = s_v[sl]
                dv = d_v[sl]
                if all_attrs and not last:
                    sg = (a - 1) * N + sv
                    sd = dv
                else:
                    if all_attrs:
                        act = None
                    else:
                        act = a <= (t + 1)
                    if last:
                        pos = blk * EBLK + ci * CHUNK + m * 16 + iota
                        pm = pos < EPW
                        act = pm if act is None else (act & pm)
                    sg = jnp.where(act, (a - 1) * N + sv, sv)
                    trash = N + ((ci * CHUNK + m * 16 + iota) % 128)
                    sd = jnp.where(act, dv, trash)
                sl2 = pl.ds(m * 16, 16)
                sgb[ci, sl2] = sg
                sdb[ci, sl2] = sd

        compute(0)
        gather(0, rows0, gsem0)

        def pair(j, carry):
            c0 = 2 * j
            compute(c0 + 1)
            gather(c0 + 1, rows1, gsem1)
            wait_gather(rows0, gsem0)
            scatter(c0, rows0)
            compute(c0 + 2)
            gather(c0 + 2, rows0, gsem0)
            wait_gather(rows1, gsem1)
            scatter(c0 + 1, rows1)
            return carry
        lax.fori_loop(0, CPB // 2 - 1, pair, 0)
        compute(CPB - 1)
        gather(CPB - 1, rows1, gsem1)
        wait_gather(rows0, gsem0)
        scatter(CPB - 2, rows0)
        wait_gather(rows1, gsem1)
        scatter(CPB - 1, rows1)

    plsc.subcore_barrier()
    pltpu.sync_copy(acc.at[pl.ds(row0, ROWS_PER_TILE)],
                    p_ref.at[c, pl.ds(row0, ROWS_PER_TILE)])


def _edge_pass(h2, attr_p, src_p, dst_p, t):
    mesh = plsc.VectorSubcoreMesh(core_axis_name="c", subcore_axis_name="s",
                                  num_cores=NC, num_subcores=NS)
    kern = pl.kernel(
        functools.partial(_edge_body, t),
        out_type=jax.ShapeDtypeStruct((NC, NACC, D), jnp.float32),
        mesh=mesh,
        scratch_types=[
            pltpu.VMEM((EBLK,), jnp.int32),
            pltpu.VMEM((EBLK,), jnp.int32),
            pltpu.VMEM((EBLK,), jnp.int32),
            pltpu.VMEM((CPB, CHUNK), jnp.int32),
            pltpu.VMEM((CPB, CHUNK), jnp.int32),
            pltpu.VMEM((CHUNK, D), jnp.float32),
            pltpu.VMEM((CHUNK, D), jnp.float32),
            pltpu.VMEM_SHARED((NACC, D), jnp.float32),
            pltpu.SemaphoreType.DMA,
            pltpu.SemaphoreType.DMA,
            pltpu.SemaphoreType.DMA,
        ],
    )
    return kern(h2, attr_p, src_p, dst_p)


def _combine_body(xs_ref, p_ref, o_ref):
    o_ref[0] = xs_ref[0] + jnp.maximum(p_ref[0] + p_ref[1], 0.0)


def _combine_next(xs_stack, p, t):
    return pl.pallas_call(
        _combine_body,
        grid=(N // BM,),
        in_specs=[
            pl.BlockSpec((1, BM, D), lambda i, t=t: (t, i, 0)),
            pl.BlockSpec((NC, BM, D), lambda i: (0, i, 0)),
        ],
        out_specs=pl.BlockSpec((1, BM, D), lambda i, t=t: (t + 1, i, 0)),
        out_shape=jax.ShapeDtypeStruct((NUM_LAYERS, N, D), jnp.float32),
        input_output_aliases={0: 0},
    )(xs_stack, p)


def _combine_final_body(xs_ref, p_ref, o_ref):
    o_ref[...] = xs_ref[0] + jnp.maximum(p_ref[0] + p_ref[1], 0.0)


def _combine_final(xs_stack, p):
    return pl.pallas_call(
        _combine_final_body,
        grid=(N // BM,),
        in_specs=[
            pl.BlockSpec((1, BM, D), lambda i: (NUM_LAYERS - 1, i, 0)),
            pl.BlockSpec((NC, BM, D), lambda i: (0, i, 0)),
        ],
        out_specs=pl.BlockSpec((BM, D), lambda i: (i, 0)),
        out_shape=jax.ShapeDtypeStruct((N, D), jnp.float32),
    )(xs_stack, p)


def kernel(x, edge_index, edge_attr, W):
    src = edge_index[0]
    dst = edge_index[1]
    zpad = jnp.zeros((PAD,), jnp.int32)
    attr_p = jnp.concatenate([edge_attr, zpad])
    src_p = jnp.concatenate([src, zpad])
    dst_p = jnp.concatenate([dst, zpad])
    xs_stack = jnp.zeros((NUM_LAYERS, N, D), jnp.float32).at[0].set(x)
    out = None
    for t in range(NUM_LAYERS):
        h = _build_h(xs_stack, W, t)
        h2 = h.reshape(NUM_LAYERS * N, D)
        p = _edge_pass(h2, attr_p, src_p, dst_p, t)
        if t < NUM_LAYERS - 1:
            xs_stack = _combine_next(xs_stack, p, t)
        else:
            out = _combine_final(xs_stack, p)
    return out


# speedup vs baseline: 31.6152x; 1.0465x over previous
import functools

import jax
import jax.numpy as jnp
from jax import lax
from jax.experimental import pallas as pl
from jax.experimental.pallas import tpu as pltpu
from jax.experimental.pallas import tpu_sc as plsc

N = 10000
E = 320000
D = 128
NUM_LAYERS = 4
TRI = (0, 1, 3, 6)

NC = 2
NS = 16
NW = NC * NS

EPW = E // NW
CHUNK = 80

CPB = 32
EBLK = CPB * CHUNK
NBLK = -(-EPW // EBLK)
PAD = NBLK * EBLK - EPW

NACC = 10240
ROWS_PER_TILE = NACC // NS

BM = 2000


def _matmul_body(x_ref, w_ref, h_ref):
    h_ref[0] = jnp.dot(x_ref[0], w_ref[0], preferred_element_type=jnp.float32)


def _build_h(xs_stack, w_all, t):
    grid = (t + 1, N // BM)
    return pl.pallas_call(
        _matmul_body,
        grid=grid,
        in_specs=[
            pl.BlockSpec((1, BM, D), lambda j, i, t=t: (t - j, i, 0)),
            pl.BlockSpec((1, D, D), lambda j, i, t=t: (TRI[t] + j, 0, 0)),
        ],
        out_specs=pl.BlockSpec((1, BM, D), lambda j, i: (j, i, 0)),
        out_shape=jax.ShapeDtypeStruct((NUM_LAYERS, N, D), jnp.float32),
    )(xs_stack, w_all)


def _build_h_old(xs_stack, w_all, t):
    grid = (t, N // BM)
    return pl.pallas_call(
        _matmul_body,
        grid=grid,
        in_specs=[
            pl.BlockSpec((1, BM, D), lambda g, i, t=t: (t - 1 - g, i, 0)),
            pl.BlockSpec((1, D, D), lambda g, i, t=t: (TRI[t] + g + 1, 0, 0)),
        ],
        out_specs=pl.BlockSpec((1, BM, D), lambda g, i: (g + 1, i, 0)),
        out_shape=jax.ShapeDtypeStruct((NUM_LAYERS, N, D), jnp.float32),
    )(xs_stack, w_all)


def _matmul_inplace_body(x_ref, w_ref, hin_ref, h_ref):
    del hin_ref
    h_ref[0] = jnp.dot(x_ref[0], w_ref[0], preferred_element_type=jnp.float32)


def _build_h_new(xs_stack, w_all, h_old, t):
    return pl.pallas_call(
        _matmul_inplace_body,
        grid=(N // BM,),
        in_specs=[
            pl.BlockSpec((1, BM, D), lambda i, t=t: (t, i, 0)),
            pl.BlockSpec((1, D, D), lambda i, t=t: (TRI[t], 0, 0)),
            pl.BlockSpec(memory_space=pl.ANY),
        ],
        out_specs=pl.BlockSpec((1, BM, D), lambda i: (0, i, 0)),
        out_shape=jax.ShapeDtypeStruct((NUM_LAYERS, N, D), jnp.float32),
        input_output_aliases={2: 0},
    )(xs_stack, w_all, h_old)


def _edge_body(t, h_ref, attr_ref, src_ref, dst_ref, p_ref,
               a_v, s_v, d_v, sgb, sdb, rows0, rows1, acc,
               lsem, gsem0, gsem1):
    c = lax.axis_index("c")
    s = lax.axis_index("s")
    wid = c * NS + s
    base0 = pl.multiple_of(wid * EPW, 16)

    zero16 = jnp.zeros((16,), jnp.float32)
    def zrow(i, carry):
        for j in range(D // 16):
            rows0[i, pl.ds(j * 16, 16)] = zero16
        return carry
    lax.fori_loop(0, CHUNK, zrow, 0)

    row0 = s * ROWS_PER_TILE
    def zacc(r, carry):
        pltpu.sync_copy(rows0, acc.at[pl.ds(row0 + r * CHUNK, CHUNK)])
        return carry
    lax.fori_loop(0, ROWS_PER_TILE // CHUNK, zacc, 0)
    plsc.subcore_barrier()

    iota = lax.broadcasted_iota(jnp.int32, (16,), 0)

    def gather(ci, rows, sem):
        return pltpu.async_copy(h_ref.at[sgb.at[ci]], rows, sem)

    def wait_gather(rows, sem):
        pltpu.make_async_copy(h_ref.at[sgb.at[0]], rows, sem).wait()

    def scatter(ci, rows):
        pltpu.sync_copy(rows, acc.at[sdb.at[ci]], add=True)

    for blk in range(NBLK):
        bbase = pl.multiple_of(base0 + blk * EBLK, 16)
        la = pltpu.async_copy(attr_ref.at[pl.ds(bbase, EBLK)], a_v, lsem)
        ls = pltpu.async_copy(src_ref.at[pl.ds(bbase, EBLK)], s_v, lsem)
        ld = pltpu.async_copy(dst_ref.at[pl.ds(bbase, EBLK)], d_v, lsem)
        la.wait()
        ls.wait()
        ld.wait()

        last = blk == NBLK - 1
        all_attrs = t == NUM_LAYERS - 1

        def compute(ci, blk=blk, last=last, all_attrs=all_attrs):
            for m in range(CHUNK // 16):
                sl = pl.ds(ci * CHUNK + m * 16, 16)
                a = a_v[sl]
                sv = s_v[sl]
                dv = d_v[sl]
                if all_attrs and not last:
                    sg = (a - 1) * N + sv
                    sd = dv
                else:
                    if all_attrs:
                        act = None
                    else:
                        act = a <= (t + 1)
                    if last:
                        pos = blk * EBLK + ci * CHUNK + m * 16 + iota
                        pm = pos < EPW
                        act = pm if act is None else (act & pm)
                    sg = jnp.where(act, (a - 1) * N + sv, sv)
                    trash = N + ((ci * CHUNK + m * 16 + iota) % 128)
                    sd = jnp.where(act, dv, trash)
                sl2 = pl.ds(m * 16, 16)
                sgb[ci, sl2] = sg
                sdb[ci, sl2] = sd

        compute(0)
        gather(0, rows0, gsem0)

        def pair(j, carry):
            c0 = 2 * j
            compute(c0 + 1)
            gather(c0 + 1, rows1, gsem1)
            wait_gather(rows0, gsem0)
            scatter(c0, rows0)
            compute(c0 + 2)
            gather(c0 + 2, rows0, gsem0)
            wait_gather(rows1, gsem1)
            scatter(c0 + 1, rows1)
            return carry
        lax.fori_loop(0, CPB // 2 - 1, pair, 0)
        compute(CPB - 1)
        gather(CPB - 1, rows1, gsem1)
        wait_gather(rows0, gsem0)
        scatter(CPB - 2, rows0)
        wait_gather(rows1, gsem1)
        scatter(CPB - 1, rows1)

    plsc.subcore_barrier()
    pltpu.sync_copy(acc.at[pl.ds(row0, ROWS_PER_TILE)],
                    p_ref.at[c, pl.ds(row0, ROWS_PER_TILE)])


def _edge_pass(h2, attr_p, src_p, dst_p, t):
    mesh = plsc.VectorSubcoreMesh(core_axis_name="c", subcore_axis_name="s",
                                  num_cores=NC, num_subcores=NS)
    kern = pl.kernel(
        functools.partial(_edge_body, t),
        out_type=jax.ShapeDtypeStruct((NC, NACC, D), jnp.float32),
        mesh=mesh,
        scratch_types=[
            pltpu.VMEM((EBLK,), jnp.int32),
            pltpu.VMEM((EBLK,), jnp.int32),
            pltpu.VMEM((EBLK,), jnp.int32),
            pltpu.VMEM((CPB, CHUNK), jnp.int32),
            pltpu.VMEM((CPB, CHUNK), jnp.int32),
            pltpu.VMEM((CHUNK, D), jnp.float32),
            pltpu.VMEM((CHUNK, D), jnp.float32),
            pltpu.VMEM_SHARED((NACC, D), jnp.float32),
            pltpu.SemaphoreType.DMA,
            pltpu.SemaphoreType.DMA,
            pltpu.SemaphoreType.DMA,
        ],
    )
    return kern(h2, attr_p, src_p, dst_p)


def _combine_body(xs_ref, p_ref, o_ref):
    o_ref[0] = xs_ref[0] + jnp.maximum(p_ref[0] + p_ref[1], 0.0)


def _combine_next(xs_stack, p, t):
    return pl.pallas_call(
        _combine_body,
        grid=(N // BM,),
        in_specs=[
            pl.BlockSpec((1, BM, D), lambda i, t=t: (t, i, 0)),
            pl.BlockSpec((NC, BM, D), lambda i: (0, i, 0)),
        ],
        out_specs=pl.BlockSpec((1, BM, D), lambda i, t=t: (t + 1, i, 0)),
        out_shape=jax.ShapeDtypeStruct((NUM_LAYERS, N, D), jnp.float32),
        input_output_aliases={0: 0},
    )(xs_stack, p)


def _combine_final_body(xs_ref, p_ref, o_ref):
    o_ref[...] = xs_ref[0] + jnp.maximum(p_ref[0] + p_ref[1], 0.0)


def _combine_final(xs_stack, p):
    return pl.pallas_call(
        _combine_final_body,
        grid=(N // BM,),
        in_specs=[
            pl.BlockSpec((1, BM, D), lambda i: (NUM_LAYERS - 1, i, 0)),
            pl.BlockSpec((NC, BM, D), lambda i: (0, i, 0)),
        ],
        out_specs=pl.BlockSpec((BM, D), lambda i: (i, 0)),
        out_shape=jax.ShapeDtypeStruct((N, D), jnp.float32),
    )(xs_stack, p)


def kernel(x, edge_index, edge_attr, W):
    src = edge_index[0]
    dst = edge_index[1]
    zpad = jnp.zeros((PAD,), jnp.int32)
    attr_p = jnp.concatenate([edge_attr, zpad])
    src_p = jnp.concatenate([src, zpad])
    dst_p = jnp.concatenate([dst, zpad])
    xs_stack = jnp.zeros((NUM_LAYERS, N, D), jnp.float32).at[0].set(x)
    out = None
    h = _build_h(xs_stack, W, 0)
    for t in range(NUM_LAYERS):
        h2 = h.reshape(NUM_LAYERS * N, D)
        p = _edge_pass(h2, attr_p, src_p, dst_p, t)
        if t < NUM_LAYERS - 1:
            h_old = _build_h_old(xs_stack, W, t + 1)
            xs_stack = _combine_next(xs_stack, p, t)
            h = _build_h_new(xs_stack, W, h_old, t + 1)
        else:
            out = _combine_final(xs_stack, p)
    return out
